# baseline (device time: 15144 ns/iter reference)
import jax
import jax.numpy as jnp
from jax import lax
from jax.experimental import pallas as pl
from jax.experimental.pallas import tpu as pltpu

MASKS = (1, 3, 4)

SCHED = ((1, 3, 4), (3, 4, 1), (4, 1, 3))
GROUPS = ((0, 176), (176, 176), (352, 160))


def kernel(t, W):
    m, k = t.shape
    _, n = W.shape
    n_rounds = len(MASKS)

    def body(t_ref, w_ref, out_ref, send_ref, recv_ref, send_sems, recv_sems):
        my_pos = lax.axis_index("i")

        barrier_sem = pltpu.get_barrier_semaphore()
        for mask in MASKS:
            pl.semaphore_signal(
                barrier_sem,
                inc=1,
                device_id=(my_pos ^ mask,),
                device_id_type=pl.DeviceIdType.MESH,
            )
        pl.semaphore_wait(barrier_sem, n_rounds)

        def make_rdma(r, g):
            off, length = GROUPS[g]
            return pltpu.make_async_remote_copy(
                src_ref=send_ref.at[r, pl.ds(off, length)],
                dst_ref=recv_ref.at[r, pl.ds(off, length)],
                send_sem=send_sems.at[r, g],
                recv_sem=recv_sems.at[r, g],
                device_id=(my_pos ^ SCHED[g][r],),
                device_id_type=pl.DeviceIdType.MESH,
            )

        for g, (off, length) in enumerate(GROUPS):
            send_ref[0, pl.ds(off, length)] = t_ref[pl.ds(off, length)].astype(
                jnp.bfloat16
            )
            make_rdma(0, g).start()

        for r in range(n_rounds):
            for g, (off, length) in enumerate(GROUPS):
                make_rdma(r, g).wait_recv()
                rows = pl.ds(off, length)
                total_g = send_ref[r, rows] + recv_ref[r, rows]
                if r + 1 < n_rounds:
                    send_ref[r + 1, rows] = total_g
                    make_rdma(r + 1, g).start()
                else:
                    out_ref[rows, :] = jnp.dot(
                        total_g,
                        w_ref[...].astype(jnp.bfloat16),
                        preferred_element_type=jnp.float32,
                    )

        for r in range(n_rounds):
            for g in range(len(GROUPS)):
                make_rdma(r, g).wait_send()

    return pl.pallas_call(
        body,
        out_shape=jax.ShapeDtypeStruct((m, n), jnp.float32),
        in_specs=[
            pl.BlockSpec(memory_space=pltpu.VMEM),
            pl.BlockSpec(memory_space=pltpu.VMEM),
        ],
        out_specs=pl.BlockSpec(memory_space=pltpu.VMEM),
        scratch_shapes=[
            pltpu.VMEM((n_rounds, m, k), jnp.bfloat16),
            pltpu.VMEM((n_rounds, m, k), jnp.bfloat16),
            pltpu.SemaphoreType.DMA((n_rounds, len(GROUPS))),
            pltpu.SemaphoreType.DMA((n_rounds, len(GROUPS))),
        ],
        compiler_params=pltpu.CompilerParams(collective_id=0),
    )(t, W)


# device time: 14864 ns/iter; 1.0188x vs baseline; 1.0188x over previous
import jax
import jax.numpy as jnp
from jax import lax
from jax.experimental import pallas as pl
from jax.experimental.pallas import tpu as pltpu

MASKS = (1, 3, 4)

SCHED = ((1, 3, 4), (3, 4, 1), (4, 1, 3))
GROUPS = ((0, 176), (176, 176), (352, 160))


def kernel(t, W):
    m, k = t.shape
    _, n = W.shape
    n_rounds = len(MASKS)

    def body(t_ref, w_ref, out_ref, send_ref, recv_ref, send_sems, recv_sems):
        my_pos = lax.axis_index("i")

        barrier_sem = pltpu.get_barrier_semaphore()
        for mask in MASKS:
            pl.semaphore_signal(
                barrier_sem,
                inc=1,
                device_id=(my_pos ^ mask,),
                device_id_type=pl.DeviceIdType.MESH,
            )
        pl.semaphore_wait(barrier_sem, n_rounds)

        def make_rdma(r, g):
            off, length = GROUPS[g]
            return pltpu.make_async_remote_copy(
                src_ref=send_ref.at[r, pl.ds(off, length)],
                dst_ref=recv_ref.at[r, pl.ds(off, length)],
                send_sem=send_sems.at[r, g],
                recv_sem=recv_sems.at[r, g],
                device_id=(my_pos ^ SCHED[g][r],),
                device_id_type=pl.DeviceIdType.MESH,
            )

        for g, (off, length) in enumerate(GROUPS):
            send_ref[0, pl.ds(off, length)] = t_ref[pl.ds(off, length)].astype(
                jnp.bfloat16
            )
            make_rdma(0, g).start()

        w_bf16 = w_ref[...].astype(jnp.bfloat16)

        for r in range(n_rounds - 1):
            for g, (off, length) in enumerate(GROUPS):
                make_rdma(r, g).wait_recv()
                rows = pl.ds(off, length)
                send_ref[r + 1, rows] = send_ref[r, rows] + recv_ref[r, rows]
                make_rdma(r + 1, g).start()

        last = n_rounds - 1
        for g in range(len(GROUPS)):
            make_rdma(last, g).wait_recv()
        out_ref[...] = jnp.dot(
            send_ref[last] + recv_ref[last],
            w_bf16,
            preferred_element_type=jnp.float32,
        )

        for r in range(n_rounds):
            for g in range(len(GROUPS)):
                make_rdma(r, g).wait_send()

    return pl.pallas_call(
        body,
        out_shape=jax.ShapeDtypeStruct((m, n), jnp.float32),
        in_specs=[
            pl.BlockSpec(memory_space=pltpu.VMEM),
            pl.BlockSpec(memory_space=pltpu.VMEM),
        ],
        out_specs=pl.BlockSpec(memory_space=pltpu.VMEM),
        scratch_shapes=[
            pltpu.VMEM((n_rounds, m, k), jnp.bfloat16),
            pltpu.VMEM((n_rounds, m, k), jnp.bfloat16),
            pltpu.SemaphoreType.DMA((n_rounds, len(GROUPS))),
            pltpu.SemaphoreType.DMA((n_rounds, len(GROUPS))),
        ],
        compiler_params=pltpu.CompilerParams(collective_id=0),
    )(t, W)
